# baseline (speedup 1.0000x reference)
import functools
import math

import jax
import jax.numpy as jnp
from jax import lax
from jax.experimental import pallas as pl
from jax.experimental.pallas import tpu as pltpu
from jax.experimental.pallas import tpu_sc as plsc

VOCAB = 1000000
HID = 128
MAXLEN = 8192
LANES = 16
VPR = HID // LANES

_info = plsc.get_sparse_core_info()
NC, NS = _info.num_cores, _info.num_subcores
NW = NC * NS

SCALE = math.sqrt(float(HID))


def _make_kernel(n_flat: int):
    assert n_flat % NW == 0
    per_w = n_flat // NW
    ch = min(256, per_w)
    assert per_w % ch == 0
    n_ch = per_w // ch

    mesh = plsc.VectorSubcoreMesh(core_axis_name="c", subcore_axis_name="s")

    @functools.partial(
        pl.kernel,
        out_type=jax.ShapeDtypeStruct((n_flat, HID), jnp.float32),
        mesh=mesh,
        scratch_types=[
            pltpu.VMEM((per_w,), jnp.int32),
            pltpu.VMEM((ch, HID), jnp.float32),
            pltpu.VMEM((ch, HID), jnp.float32),
            pltpu.SemaphoreType.DMA,
        ],
    )
    def body(tok_hbm, idx_hbm, pos_hbm, out_hbm, idx_v, rows_v, pos_v, sem):
        wid = lax.axis_index("s") * NC + lax.axis_index("c")
        base = wid * per_w
        pltpu.sync_copy(idx_hbm.at[pl.ds(base, per_w)], idx_v)

        scale = jnp.full((LANES,), SCALE, dtype=jnp.float32)

        def chunk(c, carry):
            off = c * ch
            gcp = pltpu.async_copy(
                tok_hbm.at[idx_v.at[pl.ds(off, ch)]], rows_v, sem
            )
            pos_off = lax.rem(base + off, MAXLEN)
            pltpu.sync_copy(pos_hbm.at[pl.ds(pos_off, ch)], pos_v)
            gcp.wait()

            def row(r, carry2):
                for j in range(VPR):
                    sl = pl.ds(j * LANES, LANES)
                    rows_v[r, sl] = rows_v[r, sl] * scale + pos_v[r, sl]
                return carry2

            lax.fori_loop(0, ch, row, 0, unroll=2)
            pltpu.sync_copy(rows_v, out_hbm.at[pl.ds(base + off, ch)])
            return carry

        lax.fori_loop(0, n_ch, chunk, 0)

    return body


@jax.jit
def kernel(x, token_table, pos_table):
    b, s = x.shape
    flat = jnp.reshape(x, (b * s,)).astype(jnp.int32)
    out = _make_kernel(b * s)(token_table, flat, pos_table)
    return jnp.reshape(out, (b, s, HID))


# speedup vs baseline: 1.1189x; 1.1189x over previous
import functools
import math

import jax
import jax.numpy as jnp
from jax import lax
from jax.experimental import pallas as pl
from jax.experimental.pallas import tpu as pltpu
from jax.experimental.pallas import tpu_sc as plsc

HID = 128
LANES = 16
VPR = HID // LANES

_info = plsc.get_sparse_core_info()
NC, NS = _info.num_cores, _info.num_subcores
NW = NC * NS

SCALE = math.sqrt(float(HID))


def _make_kernel(b: int, s: int):
    assert s % NW == 0
    ch = s // NW

    mesh = plsc.VectorSubcoreMesh(core_axis_name="c", subcore_axis_name="s")

    @functools.partial(
        pl.kernel,
        out_type=jax.ShapeDtypeStruct((b * s, HID), jnp.float32),
        mesh=mesh,
        scratch_types=[
            pltpu.VMEM((b * ch,), jnp.int32),
            pltpu.VMEM((ch, HID), jnp.float32),
            pltpu.VMEM((ch, HID), jnp.float32),
            pltpu.VMEM((ch, HID), jnp.float32),
            pltpu.SemaphoreType.DMA,
            pltpu.SemaphoreType.DMA,
            pltpu.SemaphoreType.DMA,
            pltpu.SemaphoreType.DMA,
        ],
    )
    def body(tok_hbm, idx_hbm, pos_hbm, out_hbm, idx_v, pos_v, buf0, buf1,
             g0, g1, o0, o1):
        wid = lax.axis_index("s") * NC + lax.axis_index("c")
        s_base = wid * ch
        bufs = (buf0, buf1)
        gsems = (g0, g1)
        osems = (o0, o1)

        for bb in range(b):
            pltpu.sync_copy(idx_hbm.at[pl.ds(bb * s + s_base, ch)],
                            idx_v.at[pl.ds(bb * ch, ch)])
        pltpu.sync_copy(pos_hbm.at[pl.ds(s_base, ch)], pos_v)

        scale = jnp.full((LANES,), SCALE, dtype=jnp.float32)

        def start_gather(bb):
            return pltpu.async_copy(
                tok_hbm.at[idx_v.at[pl.ds(bb * ch, ch)]],
                bufs[bb % 2], gsems[bb % 2])

        copies = {0: start_gather(0)}
        out_copies = {}
        for bb in range(b):
            if bb + 1 < b:
                if bb - 1 >= 0:
                    out_copies[bb - 1].wait()
                copies[bb + 1] = start_gather(bb + 1)
            copies[bb].wait()
            buf = bufs[bb % 2]

            def row(r, carry):
                for j in range(VPR):
                    sl = pl.ds(j * LANES, LANES)
                    buf[r, sl] = buf[r, sl] * scale + pos_v[r, sl]
                return carry

            lax.fori_loop(0, ch, row, 0, unroll=2)
            out_copies[bb] = pltpu.async_copy(
                buf, out_hbm.at[pl.ds(bb * s + s_base, ch)], osems[bb % 2])
        out_copies[b - 2].wait()
        out_copies[b - 1].wait()

    return body


@jax.jit
def kernel(x, token_table, pos_table):
    b, s = x.shape
    flat = jnp.reshape(x, (b * s,)).astype(jnp.int32)
    out = _make_kernel(b, s)(token_table, flat, pos_table)
    return jnp.reshape(out, (b, s, HID))


# speedup vs baseline: 1.8614x; 1.6635x over previous
import functools
import math

import jax
import jax.numpy as jnp
from jax import lax
from jax.experimental import pallas as pl
from jax.experimental.pallas import tpu as pltpu
from jax.experimental.pallas import tpu_sc as plsc

HID = 128
LANES = 16
VPR = HID // LANES

_info = plsc.get_sparse_core_info()
NC, NS = _info.num_cores, _info.num_subcores
NW = NC * NS

SCALE = math.sqrt(float(HID))


def _make_kernel(b: int, s: int):
    assert s % NW == 0
    ch = s // NW

    mesh = plsc.VectorSubcoreMesh(core_axis_name="c", subcore_axis_name="s")

    @functools.partial(
        pl.kernel,
        out_type=jax.ShapeDtypeStruct((b * s, HID), jnp.float32),
        mesh=mesh,
        scratch_types=[
            pltpu.VMEM((b * ch,), jnp.int32),
            pltpu.VMEM((ch, HID), jnp.float32),
            pltpu.VMEM((ch, HID), jnp.float32),
            pltpu.VMEM((ch, HID), jnp.float32),
            pltpu.SemaphoreType.DMA,
            pltpu.SemaphoreType.DMA,
            pltpu.SemaphoreType.DMA,
            pltpu.SemaphoreType.DMA,
        ],
    )
    def body(tok_hbm, idx_hbm, pos_hbm, out_hbm, idx_v, pos_v, buf0, buf1,
             g0, g1, o0, o1):
        wid = lax.axis_index("s") * NC + lax.axis_index("c")
        s_base = wid * ch
        bufs = (buf0, buf1)
        gsems = (g0, g1)
        osems = (o0, o1)

        for bb in range(b):
            pltpu.sync_copy(idx_hbm.at[pl.ds(bb * s + s_base, ch)],
                            idx_v.at[pl.ds(bb * ch, ch)])
        pltpu.sync_copy(pos_hbm.at[pl.ds(s_base, ch)], pos_v)

        scale = jnp.full((LANES,), SCALE, dtype=jnp.float32)

        def start_gather(bb):
            return pltpu.async_copy(
                tok_hbm.at[idx_v.at[pl.ds(bb * ch, ch)]],
                bufs[bb % 2], gsems[bb % 2])

        copies = {0: start_gather(0)}
        out_copies = {}
        for bb in range(b):
            if bb + 1 < b:
                if bb - 1 >= 0:
                    out_copies[bb - 1].wait()
                copies[bb + 1] = start_gather(bb + 1)
            copies[bb].wait()
            buf = bufs[bb % 2]

            @plsc.parallel_loop(0, ch, unroll=4)
            def row(r):
                for j in range(VPR):
                    sl = pl.ds(j * LANES, LANES)
                    buf[r, sl] = buf[r, sl] * scale + pos_v[r, sl]
            out_copies[bb] = pltpu.async_copy(
                buf, out_hbm.at[pl.ds(bb * s + s_base, ch)], osems[bb % 2])
        out_copies[b - 2].wait()
        out_copies[b - 1].wait()

    return body


@jax.jit
def kernel(x, token_table, pos_table):
    b, s = x.shape
    flat = jnp.reshape(x, (b * s,)).astype(jnp.int32)
    out = _make_kernel(b, s)(token_table, flat, pos_table)
    return jnp.reshape(out, (b, s, HID))


# speedup vs baseline: 1.9580x; 1.0519x over previous
import functools
import math

import jax
import jax.numpy as jnp
from jax import lax
from jax.experimental import pallas as pl
from jax.experimental.pallas import tpu as pltpu
from jax.experimental.pallas import tpu_sc as plsc

HID = 128
LANES = 16
VPR = HID // LANES

_info = plsc.get_sparse_core_info()
NC, NS = _info.num_cores, _info.num_subcores
NW = NC * NS

SCALE = math.sqrt(float(HID))
NBUF = 4


def _make_kernel(b: int, s: int):
    assert s % NW == 0
    spw = s // NW
    ch = min(128, spw)
    cpb = spw // ch
    n_ch = b * cpb

    mesh = plsc.VectorSubcoreMesh(core_axis_name="c", subcore_axis_name="s")

    @functools.partial(
        pl.kernel,
        out_type=jax.ShapeDtypeStruct((b, s, HID), jnp.float32),
        mesh=mesh,
        scratch_types=[
            pltpu.VMEM((b, spw), jnp.int32),
            pltpu.VMEM((spw, HID), jnp.float32),
            [pltpu.VMEM((ch, HID), jnp.float32)] * NBUF,
            [pltpu.SemaphoreType.DMA] * NBUF,
            [pltpu.SemaphoreType.DMA] * NBUF,
            pltpu.SemaphoreType.DMA,
            pltpu.SemaphoreType.DMA,
        ],
    )
    def body(tok_hbm, idx_hbm, pos_hbm, out_hbm, idx_v, pos_v, bufs,
             gsems, osems, isem, psem):
        wid = lax.axis_index("s") * NC + lax.axis_index("c")
        s_base = wid * spw

        icp = pltpu.async_copy(idx_hbm.at[:, pl.ds(s_base, spw)], idx_v, isem)
        pcp = pltpu.async_copy(pos_hbm.at[pl.ds(s_base, spw)], pos_v, psem)

        scale = jnp.full((LANES,), SCALE, dtype=jnp.float32)

        def start_gather(k):
            bb, h = k // cpb, k % cpb
            return pltpu.async_copy(
                tok_hbm.at[idx_v.at[bb, pl.ds(h * ch, ch)]],
                bufs[k % NBUF], gsems[k % NBUF])

        depth = min(NBUF - 1, n_ch)
        icp.wait()
        copies = {k: start_gather(k) for k in range(depth)}
        out_copies = {}
        pcp.wait()
        for k in range(n_ch):
            copies[k].wait()
            buf = bufs[k % NBUF]
            pbase = (k % cpb) * ch

            @plsc.parallel_loop(0, ch, unroll=8)
            def row(r):
                for j in range(VPR):
                    sl = pl.ds(j * LANES, LANES)
                    buf[r, sl] = buf[r, sl] * scale + pos_v[pbase + r, sl]

            bb, h = k // cpb, k % cpb
            out_copies[k] = pltpu.async_copy(
                buf, out_hbm.at[bb, pl.ds(s_base + h * ch, ch)],
                osems[k % NBUF])
            if k + depth < n_ch:
                prev = k + depth - NBUF
                if prev >= 0:
                    out_copies[prev].wait()
                copies[k + depth] = start_gather(k + depth)
        for k in range(max(0, n_ch - NBUF), n_ch):
            if k in out_copies:
                out_copies[k].wait()

    return body


@jax.jit
def kernel(x, token_table, pos_table):
    b, s = x.shape
    out = _make_kernel(b, s)(token_table, x.astype(jnp.int32), pos_table)
    return out
